# baseline (speedup 1.0000x reference)
import functools

import jax
import jax.numpy as jnp
from jax import lax
from jax.experimental import pallas as pl
from jax.experimental.pallas import tpu as pltpu
from jax.experimental.pallas import tpu_sc as plsc

BATCH = 16384
FEATS = 64
NUM_CORES = 2
NUM_SUBCORES = 16
NW = NUM_CORES * NUM_SUBCORES
B_PER_W = BATCH // NW
GROUPS = B_PER_W // 16
GCHUNK = 128
NCHUNK = B_PER_W // GCHUNK


def _body(home_hbm, away_hbm, x_hbm, s_hbm, beta_hbm, mu_hbm, out_hbm,
          hidx_v, aidx_v, sh_v, sa_v, x_v, beta_v, mu_v, out_v, t_v, sem):
    cid = lax.axis_index("c")
    sid = lax.axis_index("s")
    wid = sid * NUM_CORES + cid
    base = wid * B_PER_W

    pltpu.sync_copy(home_hbm.at[pl.ds(base, B_PER_W)], hidx_v)
    pltpu.sync_copy(away_hbm.at[pl.ds(base, B_PER_W)], aidx_v)

    copies = []
    for c in range(NCHUNK):
        sl = pl.ds(c * GCHUNK, GCHUNK)
        copies.append(pltpu.async_copy(s_hbm.at[hidx_v.at[sl]], sh_v.at[sl], sem))
        copies.append(pltpu.async_copy(s_hbm.at[aidx_v.at[sl]], sa_v.at[sl], sem))
    pltpu.sync_copy(x_hbm.at[pl.ds(wid * (B_PER_W // 2), B_PER_W // 2)], x_v)
    pltpu.sync_copy(beta_hbm, beta_v)
    pltpu.sync_copy(mu_hbm, mu_v)
    for cp in copies:
        cp.wait()

    lanes = lax.iota(jnp.int32, 16)
    mu_s = mu_v[...]
    b0 = beta_v[pl.ds(0, 16)]
    b1 = beta_v[pl.ds(16, 16)]
    b2 = beta_v[pl.ds(32, 16)]
    b3 = beta_v[pl.ds(48, 16)]
    idx15 = lanes * 16 + 15

    def group(g, carry):
        goff = g * 16
        grow = g * 8
        for j in range(16):
            row = grow + j // 2
            c0 = (j % 2) * 64
            t = (x_v[row, pl.ds(c0, 16)] * b0
                 + x_v[row, pl.ds(c0 + 16, 16)] * b1
                 + x_v[row, pl.ds(c0 + 32, 16)] * b2
                 + x_v[row, pl.ds(c0 + 48, 16)] * b3)
            t_v[pl.ds(j * 16, 16)] = plsc.cumsum(t)
        rs = plsc.load_gather(t_v, [idx15])
        out_v[pl.ds(goff, 16)] = (
            sh_v[pl.ds(goff, 16)] - sa_v[pl.ds(goff, 16)] + mu_s + rs)
        return carry

    lax.fori_loop(0, GROUPS, group, 0)
    pltpu.sync_copy(out_v, out_hbm.at[pl.ds(base, B_PER_W)])


@jax.jit
def kernel(home_idx, away_idx, X, strengths, beta, mu):
    mu16 = jnp.broadcast_to(mu, (16,))
    x2 = X.reshape(BATCH // 2, 128)
    run = functools.partial(
        pl.kernel,
        mesh=plsc.VectorSubcoreMesh(core_axis_name="c", subcore_axis_name="s"),
        out_type=jax.ShapeDtypeStruct((BATCH,), jnp.float32),
        compiler_params=pltpu.CompilerParams(needs_layout_passes=False),
        scratch_types=[
            pltpu.VMEM((B_PER_W,), jnp.int32),
            pltpu.VMEM((B_PER_W,), jnp.int32),
            pltpu.VMEM((B_PER_W,), jnp.float32),
            pltpu.VMEM((B_PER_W,), jnp.float32),
            pltpu.VMEM((B_PER_W // 2, 128), jnp.float32),
            pltpu.VMEM((FEATS,), jnp.float32),
            pltpu.VMEM((16,), jnp.float32),
            pltpu.VMEM((B_PER_W,), jnp.float32),
            pltpu.VMEM((256,), jnp.float32),
            pltpu.SemaphoreType.DMA,
        ],
    )(_body)
    return run(home_idx, away_idx, x2, strengths, beta, mu16)


# speedup vs baseline: 1.3235x; 1.3235x over previous
import functools

import jax
import jax.numpy as jnp
from jax import lax
from jax.experimental import pallas as pl
from jax.experimental.pallas import tpu as pltpu
from jax.experimental.pallas import tpu_sc as plsc

BATCH = 16384
FEATS = 64
NUM_CORES = 2
NUM_SUBCORES = 16
NW = NUM_CORES * NUM_SUBCORES
B_PER_W = BATCH // NW
GROUPS = B_PER_W // 16
GCHUNK = 128
NCHUNK = B_PER_W // GCHUNK


def _body(home_hbm, away_hbm, x_hbm, s_hbm, beta_hbm, mu_hbm, out_hbm,
          hidx_v, aidx_v, sh_v, sa_v, x_v, beta_v, mu_v, out_v, t_v, sem):
    cid = lax.axis_index("c")
    sid = lax.axis_index("s")
    wid = sid * NUM_CORES + cid
    base = wid * B_PER_W

    pltpu.sync_copy(home_hbm.at[pl.ds(base, B_PER_W)], hidx_v)
    pltpu.sync_copy(away_hbm.at[pl.ds(base, B_PER_W)], aidx_v)

    copies = []
    for c in range(NCHUNK):
        sl = pl.ds(c * GCHUNK, GCHUNK)
        copies.append(pltpu.async_copy(s_hbm.at[hidx_v.at[sl]], sh_v.at[sl], sem))
        copies.append(pltpu.async_copy(s_hbm.at[aidx_v.at[sl]], sa_v.at[sl], sem))
    pltpu.sync_copy(x_hbm.at[pl.ds(base, B_PER_W)], x_v)
    pltpu.sync_copy(beta_hbm, beta_v)
    pltpu.sync_copy(mu_hbm, mu_v)
    for cp in copies:
        cp.wait()

    lanes = lax.iota(jnp.int32, 16)
    mu_s = mu_v[...]
    b0 = beta_v[pl.ds(0, 16)]
    b1 = beta_v[pl.ds(16, 16)]
    b2 = beta_v[pl.ds(32, 16)]
    b3 = beta_v[pl.ds(48, 16)]
    idx15 = lanes * 16 + 15

    @plsc.parallel_loop(0, GROUPS, unroll=2)
    def group(g):
        goff = g * 16
        toff = g * 256
        for j in range(16):
            row = goff + j
            t = (x_v[row, pl.ds(0, 16)] * b0
                 + x_v[row, pl.ds(16, 16)] * b1
                 + x_v[row, pl.ds(32, 16)] * b2
                 + x_v[row, pl.ds(48, 16)] * b3)
            t_v[pl.ds(toff + j * 16, 16)] = plsc.cumsum(t)
        rs = plsc.load_gather(t_v, [toff + idx15])
        out_v[pl.ds(goff, 16)] = (
            sh_v[pl.ds(goff, 16)] - sa_v[pl.ds(goff, 16)] + mu_s + rs)
    pltpu.sync_copy(out_v, out_hbm.at[pl.ds(base, B_PER_W)])


@jax.jit
def kernel(home_idx, away_idx, X, strengths, beta, mu):
    mu16 = jnp.broadcast_to(mu, (16,))
    run = functools.partial(
        pl.kernel,
        mesh=plsc.VectorSubcoreMesh(core_axis_name="c", subcore_axis_name="s"),
        out_type=jax.ShapeDtypeStruct((BATCH,), jnp.float32),
        compiler_params=pltpu.CompilerParams(needs_layout_passes=False),
        scratch_types=[
            pltpu.VMEM((B_PER_W,), jnp.int32),
            pltpu.VMEM((B_PER_W,), jnp.int32),
            pltpu.VMEM((B_PER_W,), jnp.float32),
            pltpu.VMEM((B_PER_W,), jnp.float32),
            pltpu.VMEM((B_PER_W, FEATS), jnp.float32),
            pltpu.VMEM((FEATS,), jnp.float32),
            pltpu.VMEM((16,), jnp.float32),
            pltpu.VMEM((B_PER_W,), jnp.float32),
            pltpu.VMEM((B_PER_W * 16,), jnp.float32),
            pltpu.SemaphoreType.DMA,
        ],
    )(_body)
    return run(home_idx, away_idx, X, strengths, beta, mu16)


# speedup vs baseline: 1.3296x; 1.0046x over previous
import functools

import jax
import jax.numpy as jnp
from jax import lax
from jax.experimental import pallas as pl
from jax.experimental.pallas import tpu as pltpu
from jax.experimental.pallas import tpu_sc as plsc

BATCH = 16384
FEATS = 64
NUM_CORES = 2
NUM_SUBCORES = 16
NW = NUM_CORES * NUM_SUBCORES
B_PER_W = BATCH // NW
GROUPS = B_PER_W // 16
GCHUNK = 128
NCHUNK = B_PER_W // GCHUNK


def _body(home_hbm, away_hbm, x_hbm, s_hbm, beta_hbm, mu_hbm, out_hbm,
          hidx_v, aidx_v, sh_v, sa_v, x_v, beta_v, mu_v, out_v, t_v, sem):
    cid = lax.axis_index("c")
    sid = lax.axis_index("s")
    wid = sid * NUM_CORES + cid
    base = wid * B_PER_W

    pltpu.sync_copy(home_hbm.at[pl.ds(base, B_PER_W)], hidx_v)
    pltpu.sync_copy(away_hbm.at[pl.ds(base, B_PER_W)], aidx_v)

    copies = []
    for c in range(NCHUNK):
        sl = pl.ds(c * GCHUNK, GCHUNK)
        copies.append(pltpu.async_copy(s_hbm.at[hidx_v.at[sl]], sh_v.at[sl], sem))
        copies.append(pltpu.async_copy(s_hbm.at[aidx_v.at[sl]], sa_v.at[sl], sem))
    pltpu.sync_copy(x_hbm.at[pl.ds(base, B_PER_W)], x_v)
    pltpu.sync_copy(beta_hbm, beta_v)
    pltpu.sync_copy(mu_hbm, mu_v)
    for cp in copies:
        cp.wait()

    lanes = lax.iota(jnp.int32, 16)
    mu_s = mu_v[...]
    b0 = beta_v[pl.ds(0, 16)]
    b1 = beta_v[pl.ds(16, 16)]
    b2 = beta_v[pl.ds(32, 16)]
    b3 = beta_v[pl.ds(48, 16)]
    idx15 = lanes * 16 + 15

    @plsc.parallel_loop(0, GROUPS, unroll=2)
    def group(g):
        goff = g * 16
        toff = g * 256
        for j in range(16):
            row = goff + j
            t = (x_v[row, pl.ds(0, 16)] * b0
                 + x_v[row, pl.ds(16, 16)] * b1
                 + x_v[row, pl.ds(32, 16)] * b2
                 + x_v[row, pl.ds(48, 16)] * b3)
            t_v[pl.ds(toff + j * 16, 16)] = plsc.cumsum(t)
        rs = plsc.load_gather(t_v, [toff + idx15])
        out_v[pl.ds(goff, 16)] = (
            sh_v[pl.ds(goff, 16)] - sa_v[pl.ds(goff, 16)] + mu_s + rs)
    pltpu.sync_copy(out_v, out_hbm.at[pl.ds(base, B_PER_W)])


@jax.jit
def kernel(home_idx, away_idx, X, strengths, beta, mu):
    mu16 = jnp.broadcast_to(mu, (16,))
    run = functools.partial(
        pl.kernel,
        mesh=plsc.VectorSubcoreMesh(core_axis_name="c", subcore_axis_name="s"),
        out_type=jax.ShapeDtypeStruct((BATCH,), jnp.float32),
        compiler_params=pltpu.CompilerParams(
            needs_layout_passes=False, use_tc_tiling_on_sc=True),
        scratch_types=[
            pltpu.VMEM((B_PER_W,), jnp.int32),
            pltpu.VMEM((B_PER_W,), jnp.int32),
            pltpu.VMEM((B_PER_W,), jnp.float32),
            pltpu.VMEM((B_PER_W,), jnp.float32),
            pltpu.VMEM((B_PER_W, FEATS), jnp.float32),
            pltpu.VMEM((FEATS,), jnp.float32),
            pltpu.VMEM((16,), jnp.float32),
            pltpu.VMEM((B_PER_W,), jnp.float32),
            pltpu.VMEM((B_PER_W * 16,), jnp.float32),
            pltpu.SemaphoreType.DMA,
        ],
    )(_body)
    return run(home_idx, away_idx, X, strengths, beta, mu16)


# speedup vs baseline: 1.5933x; 1.1983x over previous
import functools

import jax
import jax.numpy as jnp
from jax import lax
from jax.experimental import pallas as pl
from jax.experimental.pallas import tpu as pltpu
from jax.experimental.pallas import tpu_sc as plsc

BATCH = 16384
FEATS = 64
NUM_CORES = 2
NUM_SUBCORES = 16
NW = NUM_CORES * NUM_SUBCORES
B_PER_W = BATCH // NW
GROUPS = B_PER_W // 16
GCHUNK = 128
NCHUNK = B_PER_W // GCHUNK


def _body(home_hbm, away_hbm, xt_hbm, s_hbm, beta_hbm, mu_hbm, out_hbm,
          hidx_v, aidx_v, sh_v, sa_v, xt_v, beta_v, bsp_v, mu_v, out_v, sem):
    cid = lax.axis_index("c")
    sid = lax.axis_index("s")
    wid = sid * NUM_CORES + cid
    base = wid * B_PER_W

    pltpu.sync_copy(home_hbm.at[pl.ds(base, B_PER_W)], hidx_v)
    pltpu.sync_copy(away_hbm.at[pl.ds(base, B_PER_W)], aidx_v)

    copies = []
    for c in range(NCHUNK):
        sl = pl.ds(c * GCHUNK, GCHUNK)
        copies.append(pltpu.async_copy(s_hbm.at[hidx_v.at[sl]], sh_v.at[sl], sem))
        copies.append(pltpu.async_copy(s_hbm.at[aidx_v.at[sl]], sa_v.at[sl], sem))
    for f in range(FEATS):
        copies.append(pltpu.async_copy(
            xt_hbm.at[f, pl.ds(base, B_PER_W)], xt_v.at[f], sem))
    pltpu.sync_copy(beta_hbm, beta_v)
    pltpu.sync_copy(mu_hbm, mu_v)
    for cp in copies:
        cp.wait()

    mu_s = mu_v[...]

    for f in range(FEATS):
        bsp_v[f, :] = plsc.load_gather(
            beta_v, [jnp.full((16,), f + 8, dtype=jnp.int32)])

    zero = jnp.zeros((16,), jnp.float32)
    accs0 = (zero,) * GROUPS

    def fbody(f, accs):
        bs = bsp_v[f, :]
        return tuple(
            acc + xt_v[f, pl.ds(g * 16, 16)] * bs
            for g, acc in enumerate(accs))

    floop = lax.fori_loop(0, FEATS, fbody, accs0)

    for g in range(GROUPS):
        goff = g * 16
        out_v[pl.ds(goff, 16)] = (
            sh_v[pl.ds(goff, 16)] - sa_v[pl.ds(goff, 16)] + mu_s + floop[g])

    pltpu.sync_copy(out_v, out_hbm.at[pl.ds(base, B_PER_W)])


@jax.jit
def kernel(home_idx, away_idx, X, strengths, beta, mu):
    mu16 = jnp.broadcast_to(mu, (16,))
    beta_pad = jnp.concatenate([jnp.zeros((8,), jnp.float32), beta])
    xt = X.T
    run = functools.partial(
        pl.kernel,
        mesh=plsc.VectorSubcoreMesh(core_axis_name="c", subcore_axis_name="s"),
        out_type=jax.ShapeDtypeStruct((BATCH,), jnp.float32),
        compiler_params=pltpu.CompilerParams(needs_layout_passes=False),
        scratch_types=[
            pltpu.VMEM((B_PER_W,), jnp.int32),
            pltpu.VMEM((B_PER_W,), jnp.int32),
            pltpu.VMEM((B_PER_W,), jnp.float32),
            pltpu.VMEM((B_PER_W,), jnp.float32),
            pltpu.VMEM((FEATS, B_PER_W), jnp.float32),
            pltpu.VMEM((FEATS + 8,), jnp.float32),
            pltpu.VMEM((FEATS, 16), jnp.float32),
            pltpu.VMEM((16,), jnp.float32),
            pltpu.VMEM((B_PER_W,), jnp.float32),
            pltpu.SemaphoreType.DMA,
        ],
    )(_body)
    return run(home_idx, away_idx, xt, strengths, beta_pad, mu16)
